# Initial kernel scaffold; baseline (speedup 1.0000x reference)
import jax
import jax.numpy as jnp
from jax.experimental import pallas as pl


def kernel(x, edge_index, batch, Ws, a_src, a_dst, W_pred, b_pred):
    raise NotImplementedError("write your pallas kernel here")


# speedup vs baseline: 11.7397x; 11.7397x over previous
import functools

import jax
import jax.numpy as jnp
from jax import lax
from jax.experimental import pallas as pl
from jax.experimental.pallas import tpu as pltpu
from jax.experimental.pallas import tpu_sc as plsc

N = 10000
E = 320000
D = 128
HD = D // 2
NLAYER = 5
G = 512
T = 10

NC = 2
NS = 16
NP = 10240
RPT = NP // NS
K = 128
EP = 327680
CH = EP // NS
NCH = CH // K
RB = 1024
NB = NP // RB

_f32 = jnp.float32
_HIGH = jax.lax.Precision.HIGHEST


def _dot(a, b):
    return jax.lax.dot_general(a, b, (((1,), (0,)), ((), ())),
                               precision=_HIGH, preferred_element_type=_f32)


def _tc_layer(p, W, asrc, adst, first):
    def body(p_ref, w_ref, as_ref, ad_ref, h2_ref, es_ref, ed_ref):
        if first:
            hin = p_ref[0]
        else:
            hb = jnp.concatenate([p_ref[0], p_ref[1]], axis=1)
            hin = jnp.where(hb > 0, hb, jnp.exp(hb) - 1.0)
        h2 = _dot(hin, w_ref[...])
        h2_ref[0] = h2[:, :HD]
        h2_ref[1] = h2[:, HD:]
        es_ref[0, 0, :] = jnp.sum(h2 * as_ref[...][None, :], axis=1)
        ed_ref[0, 0, :] = jnp.sum(h2 * ad_ref[...][None, :], axis=1)

    kin, kd = p.shape[0], p.shape[2]
    return pl.pallas_call(
        body,
        grid=(NB,),
        in_specs=[
            pl.BlockSpec((kin, RB, kd), lambda i: (0, i, 0)),
            pl.BlockSpec((D, D), lambda i: (0, 0)),
            pl.BlockSpec((D,), lambda i: (0,)),
            pl.BlockSpec((D,), lambda i: (0,)),
        ],
        out_specs=[
            pl.BlockSpec((NC, RB, HD), lambda i: (0, i, 0)),
            pl.BlockSpec((1, 1, RB), lambda i: (i, 0, 0)),
            pl.BlockSpec((1, 1, RB), lambda i: (i, 0, 0)),
        ],
        out_shape=[
            jax.ShapeDtypeStruct((NC, NP, HD), _f32),
            jax.ShapeDtypeStruct((NB, 1, RB), _f32),
            jax.ShapeDtypeStruct((NB, 1, RB), _f32),
        ],
    )(p, W, asrc, adst)


def _sc_body(src_hbm, dst2_hbm, es_hbm, ed_hbm, h2f_hbm, out_hbm,
             es_v, ed_v, denv, srcb, dstb2, sidx, exstage, rows, zbuf,
             den_sh, out_sh, sem):
    cid = lax.axis_index("c")
    sid = lax.axis_index("s")
    zero16 = jnp.zeros((16,), _f32)

    pltpu.sync_copy(es_hbm, es_v)
    pltpu.sync_copy(ed_hbm, ed_v.at[pl.ds(0, N)])
    for t in range(15):
        ed_v[pl.ds(N + t * 16, 16)] = zero16
    pltpu.sync_copy(src_hbm.at[pl.ds(sid * CH, CH)], srcb)
    pltpu.sync_copy(dst2_hbm.at[pl.ds(sid * NCH, NCH)], dstb2)

    def zb(i, _):
        zbuf[pl.ds(i * 16, 16)] = zero16
        return 0

    lax.fori_loop(0, RPT // 16, zb, 0)

    def zrow(r, _):
        for c4 in range(HD // 16):
            rows[r, pl.ds(c4 * 16, 16)] = zero16
        return 0

    lax.fori_loop(0, K, zrow, 0)

    def gm(i, m):
        return jnp.maximum(m, es_v[pl.ds(i * 16, 16)])

    m = lax.fori_loop(0, N // 16, gm, jnp.full((16,), -3e38, _f32))
    lane = lax.broadcasted_iota(jnp.int32, (16,), 0)
    for sh in (8, 4, 2, 1):
        m = jnp.maximum(m, jnp.take_along_axis(m, lane ^ sh, axis=0))
    gmax = m[0]

    pltpu.sync_copy(zbuf, den_sh.at[pl.ds(sid * RPT, RPT)])
    for q in range(RPT // K):
        pltpu.sync_copy(rows, out_sh.at[pl.ds(sid * RPT + q * K, K)])
    plsc.subcore_barrier()

    def _edge_alpha(j, g, want_alpha):
        off = j * K + g * 16
        s16 = srcb[pl.ds(off, 16)]
        d16 = dstb2[j, pl.ds(g * 16, 16)]
        esg = plsc.load_gather(es_v, [s16])
        edg = plsc.load_gather(ed_v, [d16])
        z = esg + edg
        e = jnp.maximum(z, 0.2 * z)
        zc = gmax + edg
        c = jnp.maximum(zc, 0.2 * zc)
        ex = jnp.exp(e - c)
        if not want_alpha:
            return ex
        den = plsc.load_gather(denv, [d16])
        return ex / (den + 1e-16)

    def p1(j, _):
        for g in range(K // 16):
            exstage[pl.ds(g * 16, 16)] = _edge_alpha(j, g, False)
        pltpu.sync_copy(exstage, den_sh.at[dstb2.at[j]], add=True)
        return 0

    lax.fori_loop(0, NCH, p1, 0)
    plsc.subcore_barrier()
    pltpu.sync_copy(den_sh, denv)

    def p2(j, _):
        for g in range(K // 16):
            sidx[pl.ds(g * 16, 16)] = srcb[pl.ds(j * K + g * 16, 16)] + cid * NP
        pltpu.async_copy(h2f_hbm.at[sidx], rows, sem).wait()

        def scale(g, _):
            av = _edge_alpha(j, g, True)
            for ri in range(16):
                a = av[ri]
                r = g * 16 + ri
                for c4 in range(HD // 16):
                    rows[r, pl.ds(c4 * 16, 16)] = rows[r, pl.ds(c4 * 16, 16)] * a
            return 0

        lax.fori_loop(0, K // 16, scale, 0)
        pltpu.sync_copy(rows, out_sh.at[dstb2.at[j]], add=True)
        return 0

    lax.fori_loop(0, NCH, p2, 0)
    plsc.subcore_barrier()

    pltpu.sync_copy(out_sh.at[pl.ds(sid * RPT, RPT)],
                    out_hbm.at[cid, pl.ds(sid * RPT, RPT)])


_sc_layer = functools.partial(
    pl.kernel,
    out_type=jax.ShapeDtypeStruct((NC, NP, HD), _f32),
    mesh=plsc.VectorSubcoreMesh(core_axis_name="c", subcore_axis_name="s"),
    compiler_params=pltpu.CompilerParams(needs_layout_passes=False,
                                         use_tc_tiling_on_sc=False),
    scratch_types=[
        pltpu.VMEM((N,), _f32),
        pltpu.VMEM((NP,), _f32),
        pltpu.VMEM((NP,), _f32),
        pltpu.VMEM((CH,), jnp.int32),
        pltpu.VMEM((NCH, K), jnp.int32),
        pltpu.VMEM((K,), jnp.int32),
        pltpu.VMEM((K,), _f32),
        pltpu.VMEM((K, HD), _f32),
        pltpu.VMEM((RPT,), _f32),
        pltpu.VMEM_SHARED((NP,), _f32),
        pltpu.VMEM_SHARED((NP, HD), _f32),
        pltpu.SemaphoreType.DMA,
    ],
)(_sc_body)


def _tc_head(p, batch3, W_pred, b3):
    def body(p_ref, b_ref, wp_ref, bp_ref, o_ref, sums, counts):
        i = pl.program_id(0)

        @pl.when(i == 0)
        def _():
            sums[...] = jnp.zeros_like(sums)
            counts[...] = jnp.zeros_like(counts)

        hb = jnp.concatenate([p_ref[0], p_ref[1]], axis=1)
        bvec = b_ref[0, 0, :]
        row = jax.lax.broadcasted_iota(jnp.int32, (G, RB), 1) + i * RB
        gid = jax.lax.broadcasted_iota(jnp.int32, (G, RB), 0)
        oh = jnp.where((gid == bvec[None, :]) & (row < N), 1.0, 0.0).astype(_f32)
        sums[...] += _dot(oh, hb)
        counts[...] += jnp.broadcast_to(jnp.sum(oh, axis=1)[:, None], (G, D))

        @pl.when(i == NB - 1)
        def _():
            graph = sums[...] / jnp.maximum(counts[...], 1.0)
            o_ref[...] = _dot(graph, wp_ref[...]) + bp_ref[0, 0, :][None, :]

    return pl.pallas_call(
        body,
        grid=(NB,),
        in_specs=[
            pl.BlockSpec((NC, RB, HD), lambda i: (0, i, 0)),
            pl.BlockSpec((1, 1, RB), lambda i: (i, 0, 0)),
            pl.BlockSpec((D, T), lambda i: (0, 0)),
            pl.BlockSpec((1, 1, T), lambda i: (0, 0, 0)),
        ],
        out_specs=pl.BlockSpec((G, T), lambda i: (0, 0)),
        out_shape=jax.ShapeDtypeStruct((G, T), _f32),
        scratch_shapes=[
            pltpu.VMEM((G, D), _f32),
            pltpu.VMEM((G, D), _f32),
        ],
    )(p, batch3, W_pred, b3)


def kernel(x, edge_index, batch, Ws, a_src, a_dst, W_pred, b_pred):
    src = edge_index[0].astype(jnp.int32)
    dst = edge_index[1].astype(jnp.int32)
    src_p = jnp.concatenate([src, jnp.zeros((EP - E,), jnp.int32)])
    dst_p = jnp.concatenate([dst, jnp.full((EP - E,), NP - 1, jnp.int32)])
    dst2 = dst_p.reshape(EP // K, K)

    p = jnp.pad(x, ((0, NP - N), (0, 0)))[None]
    for l in range(NLAYER):
        h2s, es3, ed3 = _tc_layer(p, Ws[l], a_src[l], a_dst[l], first=(l == 0))
        es = es3.reshape(NP)[:N]
        ed = ed3.reshape(NP)[:N]
        h2f = h2s.reshape(NC * NP, HD)
        p = _sc_layer(src_p, dst2, es, ed, h2f)

    batch3 = jnp.pad(batch, (0, NP - N)).astype(jnp.int32).reshape(NB, 1, RB)
    b3 = b_pred.reshape(1, 1, T)
    return _tc_head(p, batch3, W_pred, b3)


# speedup vs baseline: 13.2139x; 1.1256x over previous
import functools

import jax
import jax.numpy as jnp
from jax import lax
from jax.experimental import pallas as pl
from jax.experimental.pallas import tpu as pltpu
from jax.experimental.pallas import tpu_sc as plsc

N = 10000
E = 320000
D = 128
HD = D // 2
NLAYER = 5
G = 512
T = 10

NC = 2
NS = 16
NP = 10240
RPT = NP // NS
K = 128
EP = 327680
CH = EP // NS
NCH = CH // K
RB = 1024
NB = NP // RB

_f32 = jnp.float32
_HIGH = jax.lax.Precision.HIGHEST


def _dot(a, b):
    return jax.lax.dot_general(a, b, (((1,), (0,)), ((), ())),
                               precision=_HIGH, preferred_element_type=_f32)


def _tc_layer(p, W, asrc, adst, first):
    def body(p_ref, w_ref, as_ref, ad_ref, h2_ref, es_ref, ed_ref):
        if first:
            hin = p_ref[0]
        else:
            hb = jnp.concatenate([p_ref[0], p_ref[1]], axis=1)
            hin = jnp.where(hb > 0, hb, jnp.exp(hb) - 1.0)
        h2 = _dot(hin, w_ref[...])
        h2_ref[0] = h2[:, :HD]
        h2_ref[1] = h2[:, HD:]
        es_ref[0, 0, :] = jnp.sum(h2 * as_ref[...][None, :], axis=1)
        ed_ref[0, 0, :] = jnp.sum(h2 * ad_ref[...][None, :], axis=1)

    kin, kd = p.shape[0], p.shape[2]
    return pl.pallas_call(
        body,
        grid=(NB,),
        in_specs=[
            pl.BlockSpec((kin, RB, kd), lambda i: (0, i, 0)),
            pl.BlockSpec((D, D), lambda i: (0, 0)),
            pl.BlockSpec((D,), lambda i: (0,)),
            pl.BlockSpec((D,), lambda i: (0,)),
        ],
        out_specs=[
            pl.BlockSpec((NC, RB, HD), lambda i: (0, i, 0)),
            pl.BlockSpec((1, 1, RB), lambda i: (i, 0, 0)),
            pl.BlockSpec((1, 1, RB), lambda i: (i, 0, 0)),
        ],
        out_shape=[
            jax.ShapeDtypeStruct((NC, NP, HD), _f32),
            jax.ShapeDtypeStruct((NB, 1, RB), _f32),
            jax.ShapeDtypeStruct((NB, 1, RB), _f32),
        ],
    )(p, W, asrc, adst)


def _sc_body(src_hbm, dst2_hbm, es_hbm, ed_hbm, h2f_hbm, out_hbm,
             es_v, ed_v, denv, srcb, dstb2, sidx0, sidx1, ex0, ex1,
             rows0, rows1, zbuf,
             den_sh, out_sh,
             gsem0, gsem1, ssem0, ssem1, sem):
    cid = lax.axis_index("c")
    sid = lax.axis_index("s")
    zero16 = jnp.zeros((16,), _f32)

    pltpu.sync_copy(es_hbm, es_v)
    pltpu.sync_copy(ed_hbm, ed_v.at[pl.ds(0, N)])
    for t in range(15):
        ed_v[pl.ds(N + t * 16, 16)] = zero16
    pltpu.sync_copy(src_hbm.at[pl.ds(sid * CH, CH)], srcb)
    pltpu.sync_copy(dst2_hbm.at[pl.ds(sid * NCH, NCH)], dstb2)

    def zb(i, _):
        zbuf[pl.ds(i * 16, 16)] = zero16
        return 0

    lax.fori_loop(0, RPT // 16, zb, 0)

    def zrow(r, _):
        for c4 in range(HD // 16):
            rows0[r, pl.ds(c4 * 16, 16)] = zero16
        return 0

    lax.fori_loop(0, K, zrow, 0)

    def gm(i, m):
        return jnp.maximum(m, es_v[pl.ds(i * 16, 16)])

    m = lax.fori_loop(0, N // 16, gm, jnp.full((16,), -3e38, _f32))
    lane = lax.broadcasted_iota(jnp.int32, (16,), 0)
    for sh in (8, 4, 2, 1):
        m = jnp.maximum(m, jnp.take_along_axis(m, lane ^ sh, axis=0))
    gmax = m[0]

    pltpu.sync_copy(zbuf, den_sh.at[pl.ds(sid * RPT, RPT)])
    for q in range(RPT // K):
        pltpu.sync_copy(rows0, out_sh.at[pl.ds(sid * RPT + q * K, K)])
    plsc.subcore_barrier()

    def _edge_alpha(j, g, want_alpha):
        off = j * K + g * 16
        s16 = srcb[pl.ds(off, 16)]
        d16 = dstb2[j, pl.ds(g * 16, 16)]
        esg = plsc.load_gather(es_v, [s16])
        edg = plsc.load_gather(ed_v, [d16])
        z = esg + edg
        e = jnp.maximum(z, 0.2 * z)
        zc = gmax + edg
        c = jnp.maximum(zc, 0.2 * zc)
        ex = jnp.exp(e - c)
        if not want_alpha:
            return ex
        den = plsc.load_gather(denv, [d16])
        return ex / (den + 1e-16)

    def p1(t, _):
        j0 = 2 * t
        j1 = j0 + 1
        for g in range(K // 16):
            ex0[pl.ds(g * 16, 16)] = _edge_alpha(j0, g, False)
        a0 = pltpu.async_copy(ex0, den_sh.at[dstb2.at[j0]], ssem0, add=True)
        for g in range(K // 16):
            ex1[pl.ds(g * 16, 16)] = _edge_alpha(j1, g, False)
        a1 = pltpu.async_copy(ex1, den_sh.at[dstb2.at[j1]], ssem1, add=True)
        a0.wait()
        a1.wait()
        return 0

    lax.fori_loop(0, NCH // 2, p1, 0)
    plsc.subcore_barrier()
    pltpu.sync_copy(den_sh, denv)

    coff = cid * NP

    def scale(j, rows, g, _):
        av = _edge_alpha(j, g, True)
        for ri in range(16):
            a = av[ri]
            r = g * 16 + ri
            for c4 in range(HD // 16):
                rows[r, pl.ds(c4 * 16, 16)] = rows[r, pl.ds(c4 * 16, 16)] * a
        return 0

    def p2(t, _):
        j0 = 2 * t
        j1 = j0 + 1
        for g in range(K // 16):
            sidx0[pl.ds(g * 16, 16)] = srcb[pl.ds(j0 * K + g * 16, 16)] + coff
        g0 = pltpu.async_copy(h2f_hbm.at[sidx0], rows0, gsem0)
        for g in range(K // 16):
            sidx1[pl.ds(g * 16, 16)] = srcb[pl.ds(j1 * K + g * 16, 16)] + coff
        g1 = pltpu.async_copy(h2f_hbm.at[sidx1], rows1, gsem1)
        g0.wait()
        lax.fori_loop(0, K // 16, functools.partial(scale, j0, rows0), 0)
        s0 = pltpu.async_copy(rows0, out_sh.at[dstb2.at[j0]], ssem0, add=True)
        g1.wait()
        lax.fori_loop(0, K // 16, functools.partial(scale, j1, rows1), 0)
        s1 = pltpu.async_copy(rows1, out_sh.at[dstb2.at[j1]], ssem1, add=True)
        s0.wait()
        s1.wait()
        return 0

    lax.fori_loop(0, NCH // 2, p2, 0)
    plsc.subcore_barrier()

    pltpu.sync_copy(out_sh.at[pl.ds(sid * RPT, RPT)],
                    out_hbm.at[cid, pl.ds(sid * RPT, RPT)])


_sc_layer = functools.partial(
    pl.kernel,
    out_type=jax.ShapeDtypeStruct((NC, NP, HD), _f32),
    mesh=plsc.VectorSubcoreMesh(core_axis_name="c", subcore_axis_name="s"),
    compiler_params=pltpu.CompilerParams(needs_layout_passes=False,
                                         use_tc_tiling_on_sc=False),
    scratch_types=[
        pltpu.VMEM((N,), _f32),
        pltpu.VMEM((NP,), _f32),
        pltpu.VMEM((NP,), _f32),
        pltpu.VMEM((CH,), jnp.int32),
        pltpu.VMEM((NCH, K), jnp.int32),
        pltpu.VMEM((K,), jnp.int32),
        pltpu.VMEM((K,), jnp.int32),
        pltpu.VMEM((K,), _f32),
        pltpu.VMEM((K,), _f32),
        pltpu.VMEM((K, HD), _f32),
        pltpu.VMEM((K, HD), _f32),
        pltpu.VMEM((RPT,), _f32),
        pltpu.VMEM_SHARED((NP,), _f32),
        pltpu.VMEM_SHARED((NP, HD), _f32),
        pltpu.SemaphoreType.DMA,
        pltpu.SemaphoreType.DMA,
        pltpu.SemaphoreType.DMA,
        pltpu.SemaphoreType.DMA,
        pltpu.SemaphoreType.DMA,
    ],
)(_sc_body)


def _tc_head(p, batch3, W_pred, b3):
    def body(p_ref, b_ref, wp_ref, bp_ref, o_ref, sums, counts):
        i = pl.program_id(0)

        @pl.when(i == 0)
        def _():
            sums[...] = jnp.zeros_like(sums)
            counts[...] = jnp.zeros_like(counts)

        hb = jnp.concatenate([p_ref[0], p_ref[1]], axis=1)
        bvec = b_ref[0, 0, :]
        row = jax.lax.broadcasted_iota(jnp.int32, (G, RB), 1) + i * RB
        gid = jax.lax.broadcasted_iota(jnp.int32, (G, RB), 0)
        oh = jnp.where((gid == bvec[None, :]) & (row < N), 1.0, 0.0).astype(_f32)
        sums[...] += _dot(oh, hb)
        counts[...] += jnp.broadcast_to(jnp.sum(oh, axis=1)[:, None], (G, D))

        @pl.when(i == NB - 1)
        def _():
            graph = sums[...] / jnp.maximum(counts[...], 1.0)
            o_ref[...] = _dot(graph, wp_ref[...]) + bp_ref[0, 0, :][None, :]

    return pl.pallas_call(
        body,
        grid=(NB,),
        in_specs=[
            pl.BlockSpec((NC, RB, HD), lambda i: (0, i, 0)),
            pl.BlockSpec((1, 1, RB), lambda i: (i, 0, 0)),
            pl.BlockSpec((D, T), lambda i: (0, 0)),
            pl.BlockSpec((1, 1, T), lambda i: (0, 0, 0)),
        ],
        out_specs=pl.BlockSpec((G, T), lambda i: (0, 0)),
        out_shape=jax.ShapeDtypeStruct((G, T), _f32),
        scratch_shapes=[
            pltpu.VMEM((G, D), _f32),
            pltpu.VMEM((G, D), _f32),
        ],
    )(p, batch3, W_pred, b3)


def kernel(x, edge_index, batch, Ws, a_src, a_dst, W_pred, b_pred):
    src = edge_index[0].astype(jnp.int32)
    dst = edge_index[1].astype(jnp.int32)
    src_p = jnp.concatenate([src, jnp.zeros((EP - E,), jnp.int32)])
    dst_p = jnp.concatenate([dst, jnp.full((EP - E,), NP - 1, jnp.int32)])
    dst2 = dst_p.reshape(EP // K, K)

    p = jnp.pad(x, ((0, NP - N), (0, 0)))[None]
    for l in range(NLAYER):
        h2s, es3, ed3 = _tc_layer(p, Ws[l], a_src[l], a_dst[l], first=(l == 0))
        es = es3.reshape(NP)[:N]
        ed = ed3.reshape(NP)[:N]
        h2f = h2s.reshape(NC * NP, HD)
        p = _sc_layer(src_p, dst2, es, ed, h2f)

    batch3 = jnp.pad(batch, (0, NP - N)).astype(jnp.int32).reshape(NB, 1, RB)
    b3 = b_pred.reshape(1, 1, T)
    return _tc_head(p, batch3, W_pred, b3)
